# baseline (speedup 1.0000x reference)
import dataclasses
import functools

import jax
import jax.numpy as jnp
from jax import lax
from jax.experimental import pallas as pl
from jax.experimental.pallas import tpu as pltpu
from jax.experimental.pallas import tpu_sc as plsc

N = 10000
E = 320000
D = 128
NC = 2
NS = 16
NW = NC * NS
E_SUB = E // NW
CHUNK = 64
CHUNKS = 156
TAIL = E_SUB - CHUNKS * CHUNK
TAIL_OFF = CHUNKS * CHUNK
NACC = 10112
ROWS_PER_SUB = NACC // NS

_mesh = plsc.VectorSubcoreMesh(core_axis_name="c", subcore_axis_name="s")

_cp = pltpu.CompilerParams()
if "needs_layout_passes" in pltpu.CompilerParams.__dataclass_fields__:
    _cp = dataclasses.replace(_cp, needs_layout_passes=False)


@functools.partial(
    pl.kernel,
    out_type=jax.ShapeDtypeStruct((NW, N), jnp.float32),
    mesh=_mesh,
    scratch_types=[
        pltpu.VMEM((E_SUB,), jnp.int32),
        pltpu.VMEM((N,), jnp.float32),
    ],
    compiler_params=_cp,
)
def _sc_degree_hist(dst_hbm, out_hbm, dstv, hist):
    wid = lax.axis_index("s") * NC + lax.axis_index("c")
    pltpu.sync_copy(dst_hbm.at[wid], dstv)

    @pl.loop(0, N, step=16)
    def _(i):
        hist[pl.ds(i, 16)] = jnp.zeros((16,), jnp.float32)

    ones = jnp.ones((16,), jnp.float32)

    @pl.loop(0, E_SUB, step=16)
    def _(i):
        idx = dstv[pl.ds(i, 16)]
        plsc.addupdate_scatter(hist, [idx], ones)

    pltpu.sync_copy(hist, out_hbm.at[wid])


@functools.partial(
    pl.kernel,
    out_type=jax.ShapeDtypeStruct((NC, NACC, D), jnp.float32),
    mesh=_mesh,
    scratch_types=[
        pltpu.VMEM((E_SUB,), jnp.int32),
        pltpu.VMEM((E_SUB,), jnp.int32),
        pltpu.VMEM((CHUNK, D), jnp.float32),
        pltpu.VMEM((CHUNK, D), jnp.float32),
        pltpu.VMEM((TAIL, D), jnp.float32),
        pltpu.VMEM_SHARED((NACC, D), jnp.float32),
        pltpu.SemaphoreType.DMA,
        pltpu.SemaphoreType.DMA,
    ],
)
def _sc_scatter_accum(
    src_hbm, dst_hbm, g_hbm, z_hbm, out_hbm,
    srcv, dstv, bufa, bufb, buft, acc, sema, semb,
):
    c = lax.axis_index("c")
    s = lax.axis_index("s")
    wid = s * NC + c
    pltpu.sync_copy(src_hbm.at[wid], srcv)
    pltpu.sync_copy(dst_hbm.at[wid], dstv)
    pltpu.sync_copy(z_hbm, acc.at[pl.ds(s * ROWS_PER_SUB, ROWS_PER_SUB)])
    plsc.subcore_barrier()

    pltpu.async_copy(g_hbm.at[srcv.at[pl.ds(0, CHUNK)]], bufa, sema)

    @pl.loop(0, CHUNKS, step=2)
    def _(j):
        pltpu.make_async_copy(
            g_hbm.at[srcv.at[pl.ds(j * CHUNK, CHUNK)]], bufa, sema
        ).wait()
        pltpu.async_copy(
            g_hbm.at[srcv.at[pl.ds((j + 1) * CHUNK, CHUNK)]], bufb, semb
        )
        pltpu.sync_copy(
            bufa, acc.at[dstv.at[pl.ds(j * CHUNK, CHUNK)]], add=True
        )
        pltpu.make_async_copy(
            g_hbm.at[srcv.at[pl.ds((j + 1) * CHUNK, CHUNK)]], bufb, semb
        ).wait()

        @pl.when(j + 2 < CHUNKS)
        def _():
            pltpu.async_copy(
                g_hbm.at[srcv.at[pl.ds((j + 2) * CHUNK, CHUNK)]], bufa, sema
            )

        pltpu.sync_copy(
            bufb, acc.at[dstv.at[pl.ds((j + 1) * CHUNK, CHUNK)]], add=True
        )

    pltpu.async_copy(g_hbm.at[srcv.at[pl.ds(TAIL_OFF, TAIL)]], buft, sema)
    pltpu.make_async_copy(g_hbm.at[srcv.at[pl.ds(TAIL_OFF, TAIL)]], buft, sema).wait()
    pltpu.sync_copy(buft, acc.at[dstv.at[pl.ds(TAIL_OFF, TAIL)]], add=True)

    plsc.subcore_barrier()
    pltpu.sync_copy(
        acc.at[pl.ds(s * ROWS_PER_SUB, ROWS_PER_SUB)],
        out_hbm.at[c, pl.ds(s * ROWS_PER_SUB, ROWS_PER_SUB)],
    )


def _tc_scale_body(degp_ref, h_ref, g_ref, dinv_ref):
    deg = jnp.sum(degp_ref[...], axis=0) + 1.0
    dinv = lax.rsqrt(deg)[:, None]
    dinv_ref[...] = dinv
    g_ref[...] = h_ref[...] * dinv


def _tc_scale(deg_part, h):
    return pl.pallas_call(
        _tc_scale_body,
        out_shape=[
            jax.ShapeDtypeStruct((N, D), jnp.float32),
            jax.ShapeDtypeStruct((N, 1), jnp.float32),
        ],
    )(deg_part, h)


def _tc_combine_body(sp_ref, g_ref, dinv_ref, w_ref, b_ref, out_ref):
    agg = sp_ref[0, :N] + sp_ref[1, :N] + g_ref[...]
    agg = agg * dinv_ref[...]
    out_ref[...] = (
        lax.dot_general(
            agg,
            w_ref[...],
            (((1,), (1,)), ((), ())),
            precision=lax.Precision.HIGHEST,
            preferred_element_type=jnp.float32,
        )
        + b_ref[...][None, :]
    )


def _tc_combine(s_part, g, dinv, w, b):
    return pl.pallas_call(
        _tc_combine_body,
        out_shape=jax.ShapeDtypeStruct((N, D), jnp.float32),
    )(s_part, g, dinv, w, b)


def kernel(H, edge_index, W, b):
    src = edge_index[0].reshape(NW, E_SUB)
    dst = edge_index[1].reshape(NW, E_SUB)
    zeros = jnp.zeros((ROWS_PER_SUB, D), jnp.float32)

    deg_part = _sc_degree_hist(dst)
    g, dinv = _tc_scale(deg_part, H)
    s_part = _sc_scatter_accum(src, dst, g, zeros)
    return _tc_combine(s_part, g, dinv, W, b)


# speedup vs baseline: 1.1488x; 1.1488x over previous
import dataclasses
import functools

import jax
import jax.numpy as jnp
from jax import lax
from jax.experimental import pallas as pl
from jax.experimental.pallas import tpu as pltpu
from jax.experimental.pallas import tpu_sc as plsc

N = 10000
E = 320000
D = 128
NC = 2
NS = 16
NW = NC * NS
E_SUB = E // NW
CHUNK = 96
CHUNKS = 104
TAIL = E_SUB - CHUNKS * CHUNK
TAIL_OFF = CHUNKS * CHUNK
NACC = 10112
ROWS_PER_SUB = NACC // NS

_mesh = plsc.VectorSubcoreMesh(core_axis_name="c", subcore_axis_name="s")

_cp = pltpu.CompilerParams()
if "needs_layout_passes" in pltpu.CompilerParams.__dataclass_fields__:
    _cp = dataclasses.replace(_cp, needs_layout_passes=False)


@functools.partial(
    pl.kernel,
    out_type=jax.ShapeDtypeStruct((NW, N), jnp.float32),
    mesh=_mesh,
    scratch_types=[
        pltpu.VMEM((E_SUB,), jnp.int32),
        pltpu.VMEM((N,), jnp.float32),
    ],
    compiler_params=_cp,
)
def _sc_degree_hist(dst_hbm, out_hbm, dstv, hist):
    wid = lax.axis_index("s") * NC + lax.axis_index("c")
    pltpu.sync_copy(dst_hbm.at[wid], dstv)

    @pl.loop(0, N, step=16)
    def _(i):
        hist[pl.ds(i, 16)] = jnp.zeros((16,), jnp.float32)

    ones = jnp.ones((16,), jnp.float32)

    @pl.loop(0, E_SUB, step=16)
    def _(i):
        idx = dstv[pl.ds(i, 16)]
        plsc.addupdate_scatter(hist, [idx], ones)

    pltpu.sync_copy(hist, out_hbm.at[wid])


@functools.partial(
    pl.kernel,
    out_type=jax.ShapeDtypeStruct((NC, NACC, D), jnp.float32),
    mesh=_mesh,
    scratch_types=[
        pltpu.VMEM((E_SUB,), jnp.int32),
        pltpu.VMEM((E_SUB,), jnp.int32),
        pltpu.VMEM((CHUNK, D), jnp.float32),
        pltpu.VMEM((CHUNK, D), jnp.float32),
        pltpu.VMEM((TAIL, D), jnp.float32),
        pltpu.VMEM_SHARED((NACC, D), jnp.float32),
        pltpu.SemaphoreType.DMA,
        pltpu.SemaphoreType.DMA,
    ],
)
def _sc_scatter_accum(
    src_hbm, dst_hbm, g_hbm, z_hbm, out_hbm,
    srcv, dstv, bufa, bufb, buft, acc, sema, semb,
):
    c = lax.axis_index("c")
    s = lax.axis_index("s")
    wid = s * NC + c
    pltpu.sync_copy(src_hbm.at[wid], srcv)
    pltpu.sync_copy(dst_hbm.at[wid], dstv)
    pltpu.sync_copy(z_hbm, acc.at[pl.ds(s * ROWS_PER_SUB, ROWS_PER_SUB)])
    plsc.subcore_barrier()

    pltpu.async_copy(g_hbm.at[srcv.at[pl.ds(0, CHUNK)]], bufa, sema)

    @pl.loop(0, CHUNKS, step=2)
    def _(j):
        pltpu.make_async_copy(
            g_hbm.at[srcv.at[pl.ds(j * CHUNK, CHUNK)]], bufa, sema
        ).wait()
        pltpu.async_copy(
            g_hbm.at[srcv.at[pl.ds((j + 1) * CHUNK, CHUNK)]], bufb, semb
        )
        pltpu.sync_copy(
            bufa, acc.at[dstv.at[pl.ds(j * CHUNK, CHUNK)]], add=True
        )
        pltpu.make_async_copy(
            g_hbm.at[srcv.at[pl.ds((j + 1) * CHUNK, CHUNK)]], bufb, semb
        ).wait()

        @pl.when(j + 2 < CHUNKS)
        def _():
            pltpu.async_copy(
                g_hbm.at[srcv.at[pl.ds((j + 2) * CHUNK, CHUNK)]], bufa, sema
            )

        pltpu.sync_copy(
            bufb, acc.at[dstv.at[pl.ds((j + 1) * CHUNK, CHUNK)]], add=True
        )

    pltpu.async_copy(g_hbm.at[srcv.at[pl.ds(TAIL_OFF, TAIL)]], buft, sema)
    pltpu.make_async_copy(g_hbm.at[srcv.at[pl.ds(TAIL_OFF, TAIL)]], buft, sema).wait()
    pltpu.sync_copy(buft, acc.at[dstv.at[pl.ds(TAIL_OFF, TAIL)]], add=True)

    plsc.subcore_barrier()
    pltpu.sync_copy(
        acc.at[pl.ds(s * ROWS_PER_SUB, ROWS_PER_SUB)],
        out_hbm.at[c, pl.ds(s * ROWS_PER_SUB, ROWS_PER_SUB)],
    )


def _tc_scale_body(degp_ref, h_ref, g_ref, dinv_ref):
    deg = jnp.sum(degp_ref[...], axis=0) + 1.0
    dinv = lax.rsqrt(deg)[:, None]
    dinv_ref[...] = dinv
    g_ref[...] = h_ref[...] * dinv


def _tc_scale(deg_part, h):
    return pl.pallas_call(
        _tc_scale_body,
        out_shape=[
            jax.ShapeDtypeStruct((N, D), jnp.float32),
            jax.ShapeDtypeStruct((N, 1), jnp.float32),
        ],
    )(deg_part, h)


def _tc_combine_body(sp_ref, g_ref, dinv_ref, w_ref, b_ref, out_ref):
    agg = sp_ref[0, :N] + sp_ref[1, :N] + g_ref[...]
    agg = agg * dinv_ref[...]
    out_ref[...] = (
        lax.dot_general(
            agg,
            w_ref[...],
            (((1,), (1,)), ((), ())),
            precision=lax.Precision.HIGHEST,
            preferred_element_type=jnp.float32,
        )
        + b_ref[...][None, :]
    )


def _tc_combine(s_part, g, dinv, w, b):
    return pl.pallas_call(
        _tc_combine_body,
        out_shape=jax.ShapeDtypeStruct((N, D), jnp.float32),
    )(s_part, g, dinv, w, b)


def kernel(H, edge_index, W, b):
    src = edge_index[0].reshape(NW, E_SUB)
    dst = edge_index[1].reshape(NW, E_SUB)
    zeros = jnp.zeros((ROWS_PER_SUB, D), jnp.float32)

    deg_part = _sc_degree_hist(dst)
    g, dinv = _tc_scale(deg_part, H)
    s_part = _sc_scatter_accum(src, dst, g, zeros)
    return _tc_combine(s_part, g, dinv, W, b)
